# baseline (speedup 1.0000x reference)
import functools

import jax
import jax.numpy as jnp
from jax import lax
from jax.experimental import pallas as pl
from jax.experimental.pallas import tpu as pltpu
from jax.experimental.pallas import tpu_sc as plsc

NUM_LABELED = 15080
OUT_CHANNELS = 2048
TEMP = 0.05
N_ROWS = 64

BLOCK = 1040
NB = (NUM_LABELED + BLOCK - 1) // BLOCK


def _lse_kernel(feat_ref, mem_ref, out_ref, m_ref, s_ref):
    k = pl.program_id(0)

    p = lax.dot_general(
        feat_ref[...], mem_ref[...],
        dimension_numbers=(((1,), (1,)), ((), ())),
        preferred_element_type=jnp.float32,
    ) * (1.0 / TEMP)

    col = k * BLOCK + lax.broadcasted_iota(jnp.int32, (N_ROWS, BLOCK), 1)
    neg = jnp.float32(-jnp.inf)
    pv = jnp.where(col < NUM_LABELED, p, neg)

    @pl.when(k == 0)
    def _init():
        m_ref[...] = jnp.full((N_ROWS, 1), neg, jnp.float32)
        s_ref[...] = jnp.zeros((N_ROWS, 1), jnp.float32)

    m_prev = m_ref[...]
    s_prev = s_ref[...]
    bmax = jnp.max(pv, axis=1, keepdims=True)
    m_new = jnp.maximum(m_prev, bmax)
    s_new = s_prev * jnp.exp(m_prev - m_new) + jnp.sum(
        jnp.exp(pv - m_new), axis=1, keepdims=True)
    m_ref[...] = m_new
    s_ref[...] = s_new

    @pl.when(k == NB - 1)
    def _finish():
        out_ref[...] = m_new + jnp.log(s_new)


def _lse_call(feat, memory):
    return pl.pallas_call(
        _lse_kernel,
        grid=(NB,),
        in_specs=[
            pl.BlockSpec((N_ROWS, OUT_CHANNELS), lambda k: (0, 0)),
            pl.BlockSpec((BLOCK, OUT_CHANNELS), lambda k: (k, 0)),
        ],
        out_specs=pl.BlockSpec((N_ROWS, 1), lambda k: (0, 0)),
        out_shape=jax.ShapeDtypeStruct((N_ROWS, 1), jnp.float32),
        scratch_shapes=[
            pltpu.VMEM((N_ROWS, 1), jnp.float32),
            pltpu.VMEM((N_ROWS, 1), jnp.float32),
        ],
        compiler_params=pltpu.CompilerParams(
            dimension_semantics=("arbitrary",),
        ),
    )(feat, memory)


_ROWS_PER_W = 16
_N_WORKERS = N_ROWS // _ROWS_PER_W
_NC = 2
_CHUNK = 16


def _picked_sc_body(feat_hbm, tgt_hbm, mem_hbm, out_hbm,
                    idx_v, feat_v, rows_v, accs_v, out_v, sem):
    wid = lax.axis_index("s") * _NC + lax.axis_index("c")

    @pl.when(wid < _N_WORKERS)
    def _():
        base = wid * _ROWS_PER_W
        pltpu.sync_copy(tgt_hbm.at[pl.ds(base, _ROWS_PER_W)], idx_v)
        pltpu.async_copy(mem_hbm.at[idx_v], rows_v, sem).wait()
        pltpu.sync_copy(feat_hbm.at[pl.ds(base, _ROWS_PER_W)], feat_v)

        lanes = lax.iota(jnp.int32, 16)
        out_vec = jnp.zeros((16,), jnp.float32)
        for i in range(_ROWS_PER_W):
            def chunk(c, a, i=i):
                sl = pl.ds(c * _CHUNK, _CHUNK)
                return a + feat_v[i, sl] * rows_v[i, sl]
            acc = lax.fori_loop(0, OUT_CHANNELS // _CHUNK, chunk,
                                jnp.zeros((16,), jnp.float32))
            s = acc[0]
            for k in range(1, 16):
                s = s + acc[k]
            out_vec = out_vec + jnp.where(lanes == i,
                                          jnp.full((16,), s, jnp.float32),
                                          jnp.zeros((16,), jnp.float32))
        out_v[...] = out_vec * (1.0 / TEMP)
        pltpu.sync_copy(out_v, out_hbm.at[pl.ds(base, _ROWS_PER_W)])


def _picked_call(feat, targets, memory):
    run = functools.partial(
        pl.kernel,
        mesh=plsc.VectorSubcoreMesh(core_axis_name="c", subcore_axis_name="s"),
        out_type=jax.ShapeDtypeStruct((N_ROWS,), jnp.float32),
        scratch_types=[
            pltpu.VMEM((_ROWS_PER_W,), jnp.int32),
            pltpu.VMEM((_ROWS_PER_W, OUT_CHANNELS), jnp.float32),
            pltpu.VMEM((_ROWS_PER_W, OUT_CHANNELS), jnp.float32),
            pltpu.VMEM((16, 16), jnp.float32),
            pltpu.VMEM((16,), jnp.float32),
            pltpu.SemaphoreType.DMA,
        ],
    )(_picked_sc_body)
    return run(feat, targets, memory)


@jax.jit
def _run(features, pids, memory):
    mask = pids > -1
    maskf = mask.astype(jnp.float32)
    targets = jnp.where(mask, pids, 0).astype(jnp.int32)
    feat = jnp.where(mask[:, None], features, 0.0)

    lse = _lse_call(feat, memory)[:, 0]
    picked = _picked_call(feat, targets, memory)

    return -jnp.sum((picked - lse) * maskf) / jnp.sum(maskf)


def kernel(features, gt_labels, memory):
    pids = gt_labels[..., -1].reshape(-1).astype(jnp.int32)
    return _run(features, pids, memory)


# speedup vs baseline: 1.0080x; 1.0080x over previous
import functools

import jax
import jax.numpy as jnp
from jax import lax
from jax.experimental import pallas as pl
from jax.experimental.pallas import tpu as pltpu
from jax.experimental.pallas import tpu_sc as plsc

NUM_LABELED = 15080
OUT_CHANNELS = 2048
TEMP = 0.05
N_ROWS = 64

BLOCK = 1040
NB = (NUM_LABELED + BLOCK - 1) // BLOCK


def _lse_kernel(feat_ref, mem_ref, out_ref, m_ref, s_ref):
    k = pl.program_id(0)

    p = lax.dot_general(
        feat_ref[...], mem_ref[...],
        dimension_numbers=(((1,), (1,)), ((), ())),
        preferred_element_type=jnp.float32,
    ) * (1.0 / TEMP)

    col = k * BLOCK + lax.broadcasted_iota(jnp.int32, (N_ROWS, BLOCK), 1)
    neg = jnp.float32(-jnp.inf)
    pv = jnp.where(col < NUM_LABELED, p, neg)

    @pl.when(k == 0)
    def _init():
        m_ref[...] = jnp.full((N_ROWS, 1), neg, jnp.float32)
        s_ref[...] = jnp.zeros((N_ROWS, 1), jnp.float32)

    m_prev = m_ref[...]
    s_prev = s_ref[...]
    bmax = jnp.max(pv, axis=1, keepdims=True)
    m_new = jnp.maximum(m_prev, bmax)
    s_new = s_prev * jnp.exp(m_prev - m_new) + jnp.sum(
        jnp.exp(pv - m_new), axis=1, keepdims=True)
    m_ref[...] = m_new
    s_ref[...] = s_new

    @pl.when(k == NB - 1)
    def _finish():
        out_ref[...] = m_new + jnp.log(s_new)


def _lse_call(feat, memory):
    return pl.pallas_call(
        _lse_kernel,
        grid=(NB,),
        in_specs=[
            pl.BlockSpec((N_ROWS, OUT_CHANNELS), lambda k: (0, 0)),
            pl.BlockSpec((BLOCK, OUT_CHANNELS), lambda k: (k, 0)),
        ],
        out_specs=pl.BlockSpec((N_ROWS, 1), lambda k: (0, 0)),
        out_shape=jax.ShapeDtypeStruct((N_ROWS, 1), jnp.float32),
        scratch_shapes=[
            pltpu.VMEM((N_ROWS, 1), jnp.float32),
            pltpu.VMEM((N_ROWS, 1), jnp.float32),
        ],
        compiler_params=pltpu.CompilerParams(
            dimension_semantics=("arbitrary",),
        ),
    )(feat, memory)


_N_WORKERS = 8
_ROWS_PER_W = N_ROWS // _N_WORKERS
_NC = 2
_CHUNK = 16
_UNROLL = 4


def _picked_sc_body(feat_hbm, tgt_hbm, mem_hbm, out_hbm,
                    idx_v, feat_v, rows_v, out_v, sem):
    wid = lax.axis_index("s") * _NC + lax.axis_index("c")

    @pl.when(wid < _N_WORKERS)
    def _():
        base = wid * _ROWS_PER_W
        pltpu.sync_copy(tgt_hbm.at[pl.ds(base, _ROWS_PER_W)], idx_v)
        gather = pltpu.async_copy(mem_hbm.at[idx_v], rows_v, sem)
        pltpu.sync_copy(feat_hbm.at[pl.ds(base, _ROWS_PER_W)], feat_v)
        gather.wait()

        lanes = lax.iota(jnp.int32, 16)
        out_vec = jnp.zeros((16,), jnp.float32)
        zero16 = jnp.zeros((16,), jnp.float32)
        n_iter = OUT_CHANNELS // (_CHUNK * _UNROLL)
        for i in range(_ROWS_PER_W):
            def chunk(c, accs, i=i):
                cbase = c * (_CHUNK * _UNROLL)
                return tuple(
                    accs[u] + feat_v[i, pl.ds(cbase + u * _CHUNK, _CHUNK)]
                    * rows_v[i, pl.ds(cbase + u * _CHUNK, _CHUNK)]
                    for u in range(_UNROLL))
            accs = lax.fori_loop(0, n_iter, chunk, (zero16,) * _UNROLL)
            acc = accs[0] + accs[1] + accs[2] + accs[3]
            s = acc[0]
            for k in range(1, 16):
                s = s + acc[k]
            out_vec = out_vec + jnp.where(lanes == i,
                                          jnp.full((16,), s, jnp.float32),
                                          zero16)
        out_v[...] = out_vec * (1.0 / TEMP)
        pltpu.sync_copy(out_v.at[pl.ds(0, _ROWS_PER_W)],
                        out_hbm.at[pl.ds(base, _ROWS_PER_W)])


def _picked_call(feat, targets, memory):
    run = functools.partial(
        pl.kernel,
        mesh=plsc.VectorSubcoreMesh(core_axis_name="c", subcore_axis_name="s"),
        out_type=jax.ShapeDtypeStruct((N_ROWS,), jnp.float32),
        scratch_types=[
            pltpu.VMEM((_ROWS_PER_W,), jnp.int32),
            pltpu.VMEM((_ROWS_PER_W, OUT_CHANNELS), jnp.float32),
            pltpu.VMEM((_ROWS_PER_W, OUT_CHANNELS), jnp.float32),
            pltpu.VMEM((16,), jnp.float32),
            pltpu.SemaphoreType.DMA,
        ],
    )(_picked_sc_body)
    return run(feat, targets, memory)


@jax.jit
def _run(features, pids, memory):
    mask = pids > -1
    maskf = mask.astype(jnp.float32)
    targets = jnp.where(mask, pids, 0).astype(jnp.int32)
    feat = jnp.where(mask[:, None], features, 0.0)

    lse = _lse_call(feat, memory)[:, 0]
    picked = _picked_call(feat, targets, memory)

    return -jnp.sum((picked - lse) * maskf) / jnp.sum(maskf)


def kernel(features, gt_labels, memory):
    pids = gt_labels[..., -1].reshape(-1).astype(jnp.int32)
    return _run(features, pids, memory)


# speedup vs baseline: 1.4867x; 1.4748x over previous
import functools

import jax
import jax.numpy as jnp
from jax.experimental import pallas as pl
from jax.experimental.pallas import tpu as pltpu

NUM_LABELED = 15080
OUT_CHANNELS = 2048
TEMP = 0.05
N_ROWS = 64

BLOCK = 2048
NB = (NUM_LABELED + BLOCK - 1) // BLOCK


def _loss_kernel(feat_ref, pids_ref, mem_ref, out_ref, m_ref, s_ref, p_ref):
    k = pl.program_id(0)

    pids = pids_ref[...]
    mask = pids > -1
    targets = jnp.where(mask, pids, 0)

    feat = feat_ref[...]
    feat = jnp.where(mask, feat, 0.0)

    p = jax.lax.dot_general(
        feat, mem_ref[...],
        dimension_numbers=(((1,), (1,)), ((), ())),
        preferred_element_type=jnp.float32,
        precision=jax.lax.Precision.DEFAULT,
    ) * (1.0 / TEMP)

    col = k * BLOCK + jax.lax.broadcasted_iota(jnp.int32, (N_ROWS, BLOCK), 1)
    valid = col < NUM_LABELED
    neg = jnp.float32(-jnp.inf)
    pv = jnp.where(valid, p, neg)

    hit = col == targets
    p_blk = jnp.sum(jnp.where(hit, p, 0.0), axis=1, keepdims=True)

    @pl.when(k == 0)
    def _init():
        m_ref[...] = jnp.full((N_ROWS, 1), neg, jnp.float32)
        s_ref[...] = jnp.zeros((N_ROWS, 1), jnp.float32)
        p_ref[...] = jnp.zeros((N_ROWS, 1), jnp.float32)

    m_prev = m_ref[...]
    s_prev = s_ref[...]
    bmax = jnp.max(pv, axis=1, keepdims=True)
    m_new = jnp.maximum(m_prev, bmax)
    s_new = s_prev * jnp.exp(m_prev - m_new) + jnp.sum(
        jnp.exp(pv - m_new), axis=1, keepdims=True)
    m_ref[...] = m_new
    s_ref[...] = s_new
    p_ref[...] = p_ref[...] + p_blk

    @pl.when(k == NB - 1)
    def _finish():
        lse = m_new + jnp.log(s_new)
        maskf = mask.astype(jnp.float32)
        picked = p_ref[...]
        loss = -jnp.sum((picked - lse) * maskf) / jnp.sum(maskf)
        out_ref[0, 0] = loss


@jax.jit
def _run(feat, pids2d, memory):
    out = pl.pallas_call(
        _loss_kernel,
        grid=(NB,),
        in_specs=[
            pl.BlockSpec((N_ROWS, OUT_CHANNELS), lambda k: (0, 0)),
            pl.BlockSpec((N_ROWS, 1), lambda k: (0, 0)),
            pl.BlockSpec((BLOCK, OUT_CHANNELS), lambda k: (k, 0)),
        ],
        out_specs=pl.BlockSpec(memory_space=pltpu.SMEM),
        out_shape=jax.ShapeDtypeStruct((1, 1), jnp.float32),
        scratch_shapes=[
            pltpu.VMEM((N_ROWS, 1), jnp.float32),
            pltpu.VMEM((N_ROWS, 1), jnp.float32),
            pltpu.VMEM((N_ROWS, 1), jnp.float32),
        ],
        compiler_params=pltpu.CompilerParams(
            dimension_semantics=("arbitrary",),
        ),
    )(feat, pids2d, memory)
    return out[0, 0]


def kernel(features, gt_labels, memory):
    pids = gt_labels[..., -1].reshape(-1, 1).astype(jnp.int32)
    return _run(features, pids, memory)
